# Initial kernel scaffold; baseline (speedup 1.0000x reference)
import jax
import jax.numpy as jnp
from jax.experimental import pallas as pl


def kernel(x, A, W1, b1, W2, b2):
    raise NotImplementedError("write your pallas kernel here")


# speedup vs baseline: 11.3919x; 11.3919x over previous
import functools

import jax
import jax.numpy as jnp
from jax import lax
from jax.experimental import pallas as pl
from jax.experimental.pallas import tpu as pltpu
from jax.experimental.pallas import tpu_sc as plsc

N = 10000
NP = 10240
E = 320000
C1 = 256
C2 = 128
CW = 64
NC = 2
NS = 16
K = 80
BLK = 32
CH = E // (BLK * K)
RPT = NP // NS
RCH = 128
F32 = jnp.float32

_mesh = plsc.VectorSubcoreMesh(
    core_axis_name="c", subcore_axis_name="s", num_cores=NC, num_subcores=NS)


def _zero_fill(buf, rows, width):
    z = jnp.zeros((16,), F32)

    def row(i, _):
        def col(k, _):
            buf[i, pl.ds(k * 16, 16)] = z
            return 0
        return lax.fori_loop(0, width // 16, col, 0)

    lax.fori_loop(0, rows, row, 0)


@functools.partial(
    pl.kernel,
    out_type=(jax.ShapeDtypeStruct((NP, 16), F32),) * 2,
    mesh=_mesh,
    scratch_types=[
        pltpu.VMEM((CH, K), jnp.int32),
        pltpu.VMEM((K, 16), F32),
        pltpu.VMEM((RPT, 16), F32),
        pltpu.VMEM_SHARED((NP, 16), F32),
    ],
    compiler_params=pltpu.CompilerParams(use_tc_tiling_on_sc=False),
)
def _sc_deg(dstF, deg0, deg1, idx_v, ones_v, zbuf, acc):
    c = lax.axis_index("c")
    s = lax.axis_index("s")
    wid = c * NS + s

    pltpu.sync_copy(dstF.at[wid], idx_v)

    one = jnp.ones((16,), F32)
    def fill_ones(i, _):
        ones_v[i, :] = one
        return 0
    lax.fori_loop(0, K, fill_ones, 0)
    _zero_fill(zbuf, RPT, 16)
    pltpu.sync_copy(zbuf, acc.at[pl.ds(s * RPT, RPT)])
    plsc.subcore_barrier()

    def chunk(j, _):
        pltpu.sync_copy(ones_v, acc.at[idx_v.at[j]], add=True)
        return 0
    lax.fori_loop(0, CH, chunk, 0)
    plsc.subcore_barrier()

    pltpu.sync_copy(acc.at[pl.ds(s * RPT, RPT)], zbuf)

    @pl.when(c == 0)
    def _():
        pltpu.sync_copy(zbuf, deg0.at[pl.ds(s * RPT, RPT)])

    @pl.when(c == 1)
    def _():
        pltpu.sync_copy(zbuf, deg1.at[pl.ds(s * RPT, RPT)])


@functools.partial(
    pl.kernel,
    out_type=(jax.ShapeDtypeStruct((NP, CW), F32),) * 2,
    mesh=_mesh,
    scratch_types=[
        pltpu.VMEM((2 * CH, K), jnp.int32),
        pltpu.VMEM((2 * CH, K), jnp.int32),
        pltpu.VMEM((RCH, CW), F32),
        pltpu.VMEM((K, CW), F32),
        pltpu.VMEM_SHARED((NP, CW), F32),
        pltpu.SemaphoreType.DMA,
    ],
    compiler_params=pltpu.CompilerParams(use_tc_tiling_on_sc=False),
)
def _sc_agg(srcF, dstF, gA, gB, outA, outB, src_v, dst_v, zbuf, rows_v, acc,
            sem):
    c = lax.axis_index("c")
    s = lax.axis_index("s")

    pltpu.sync_copy(srcF.at[s], src_v.at[pl.ds(0, CH)])
    pltpu.sync_copy(srcF.at[s + NS], src_v.at[pl.ds(CH, CH)])
    pltpu.sync_copy(dstF.at[s], dst_v.at[pl.ds(0, CH)])
    pltpu.sync_copy(dstF.at[s + NS], dst_v.at[pl.ds(CH, CH)])

    _zero_fill(zbuf, RCH, CW)
    for k in range(RPT // RCH):
        pltpu.sync_copy(zbuf, acc.at[pl.ds(s * RPT + k * RCH, RCH)])
    plsc.subcore_barrier()

    def run(g_ref):
        def chunk(j, _):
            pltpu.async_copy(g_ref.at[src_v.at[j]], rows_v, sem).wait()
            pltpu.sync_copy(rows_v, acc.at[dst_v.at[j]], add=True)
            return 0
        lax.fori_loop(0, 2 * CH, chunk, 0)

    @pl.when(c == 0)
    def _():
        run(gA)

    @pl.when(c == 1)
    def _():
        run(gB)

    plsc.subcore_barrier()

    for k in range(RPT // RCH):
        sl = pl.ds(s * RPT + k * RCH, RCH)
        pltpu.sync_copy(acc.at[sl], zbuf)

        @pl.when(c == 0)
        def _():
            pltpu.sync_copy(zbuf, outA.at[sl])

        @pl.when(c == 1)
        def _():
            pltpu.sync_copy(zbuf, outB.at[sl])


_BN = 1000


def _dinv(d0, d1):
    deg = d0[:, 0:1] + d1[:, 0:1] + 1.0
    return lax.rsqrt(deg)


def _tc_a_body(d0, d1, x_ref, w_ref, g0, g1, g2, g3):
    dinv = _dinv(d0[...], d1[...])
    h = jnp.dot(x_ref[...], w_ref[...], preferred_element_type=F32)
    g = h * dinv
    g0[...] = g[:, 0 * CW:1 * CW]
    g1[...] = g[:, 1 * CW:2 * CW]
    g2[...] = g[:, 2 * CW:3 * CW]
    g3[...] = g[:, 3 * CW:4 * CW]


def _tc_a(deg0, deg1, x, W1):
    blk = lambda w: pl.BlockSpec((_BN, w), lambda i: (i, 0))
    return pl.pallas_call(
        _tc_a_body,
        grid=(N // _BN,),
        in_specs=[
            blk(16), blk(16), blk(C2),
            pl.BlockSpec((C2, C1), lambda i: (0, 0)),
        ],
        out_specs=[blk(CW)] * 4,
        out_shape=(jax.ShapeDtypeStruct((N, CW), F32),) * 4,
    )(deg0, deg1, x, W1)


def _tc_b_body(d0, d1, s0, s1, s2, s3, g0, g1, g2, g3, b1, w2, o0, o1):
    dinv = _dinv(d0[...], d1[...])
    t = jnp.concatenate(
        [s0[...] + g0[...], s1[...] + g1[...],
         s2[...] + g2[...], s3[...] + g3[...]], axis=1)
    out1 = jnp.maximum(t * dinv + b1[...], 0.0)
    gg = jnp.dot(out1, w2[...], preferred_element_type=F32) * dinv
    o0[...] = gg[:, :CW]
    o1[...] = gg[:, CW:]


def _tc_b(deg0, deg1, s_parts, g_parts, b1, W2):
    blk = lambda w: pl.BlockSpec((_BN, w), lambda i: (i, 0))
    return pl.pallas_call(
        _tc_b_body,
        grid=(N // _BN,),
        in_specs=[blk(16), blk(16)] + [blk(CW)] * 8 + [
            pl.BlockSpec((1, C1), lambda i: (0, 0)),
            pl.BlockSpec((C1, C2), lambda i: (0, 0)),
        ],
        out_specs=[blk(CW)] * 2,
        out_shape=(jax.ShapeDtypeStruct((N, CW), F32),) * 2,
    )(deg0, deg1, *s_parts, *g_parts, b1, W2)


def _tc_c_body(d0, d1, t0, t1, g0, g1, b2, out_ref):
    dinv = _dinv(d0[...], d1[...])
    t = jnp.concatenate([t0[...] + g0[...], t1[...] + g1[...]], axis=1)
    out_ref[...] = jnp.maximum(t * dinv + b2[...], 0.0)


def _tc_c(deg0, deg1, t0, t1, g0, g1, b2):
    blk = lambda w: pl.BlockSpec((_BN, w), lambda i: (i, 0))
    return pl.pallas_call(
        _tc_c_body,
        grid=(N // _BN,),
        in_specs=[blk(16), blk(16), blk(CW), blk(CW), blk(CW), blk(CW),
                  pl.BlockSpec((1, C2), lambda i: (0, 0))],
        out_specs=blk(C2),
        out_shape=jax.ShapeDtypeStruct((N, C2), F32),
    )(deg0, deg1, t0, t1, g0, g1, b2)


def kernel(x, A, W1, b1, W2, b2):
    src = A[0].astype(jnp.int32)
    dst = A[1].astype(jnp.int32)
    srcF = src.reshape(BLK, CH, K)
    dstF = dst.reshape(BLK, CH, K)

    deg0, deg1 = _sc_deg(dstF)
    g1_0, g1_1, g1_2, g1_3 = _tc_a(deg0, deg1, x, W1)
    s0, s1 = _sc_agg(srcF, dstF, g1_0, g1_1)
    s2, s3 = _sc_agg(srcF, dstF, g1_2, g1_3)
    g2_0, g2_1 = _tc_b(deg0, deg1, (s0, s1, s2, s3),
                       (g1_0, g1_1, g1_2, g1_3), b1.reshape(1, C1), W2)
    t0, t1 = _sc_agg(srcF, dstF, g2_0, g2_1)
    return _tc_c(deg0, deg1, t0, t1, g2_0, g2_1, b2.reshape(1, C2))


# speedup vs baseline: 18.6751x; 1.6393x over previous
import functools

import jax
import jax.numpy as jnp
from jax import lax
from jax.experimental import pallas as pl
from jax.experimental.pallas import tpu as pltpu
from jax.experimental.pallas import tpu_sc as plsc

N = 10000
NP = 10240
E = 320000
C1 = 256
C2 = 128
CW = 64
NC = 2
NS = 16
K = 80
BLK = 32
CH = E // (BLK * K)
RPT = NP // NS
RCH = 128
F32 = jnp.float32

_mesh = plsc.VectorSubcoreMesh(
    core_axis_name="c", subcore_axis_name="s", num_cores=NC, num_subcores=NS)


def _zero_fill(buf, rows, width):
    z = jnp.zeros((16,), F32)

    def row(i, _):
        def col(k, _):
            buf[i, pl.ds(k * 16, 16)] = z
            return 0
        return lax.fori_loop(0, width // 16, col, 0)

    lax.fori_loop(0, rows, row, 0)


@functools.partial(
    pl.kernel,
    out_type=(jax.ShapeDtypeStruct((NP, 16), F32),) * 2,
    mesh=_mesh,
    scratch_types=[
        pltpu.VMEM((CH, K), jnp.int32),
        pltpu.VMEM((K, 16), F32),
        pltpu.VMEM((RPT, 16), F32),
        pltpu.VMEM_SHARED((NP, 16), F32),
    ],
    compiler_params=pltpu.CompilerParams(use_tc_tiling_on_sc=False),
)
def _sc_deg(dstF, deg0, deg1, idx_v, ones_v, zbuf, acc):
    c = lax.axis_index("c")
    s = lax.axis_index("s")
    wid = c * NS + s

    pltpu.sync_copy(dstF.at[wid], idx_v)

    one = jnp.ones((16,), F32)
    def fill_ones(i, _):
        ones_v[i, :] = one
        return 0
    lax.fori_loop(0, K, fill_ones, 0)
    _zero_fill(zbuf, RPT, 16)
    pltpu.sync_copy(zbuf, acc.at[pl.ds(s * RPT, RPT)])
    plsc.subcore_barrier()

    def chunk(j, _):
        pltpu.sync_copy(ones_v, acc.at[idx_v.at[j]], add=True)
        return 0
    lax.fori_loop(0, CH, chunk, 0)
    plsc.subcore_barrier()

    pltpu.sync_copy(acc.at[pl.ds(s * RPT, RPT)], zbuf)

    @pl.when(c == 0)
    def _():
        pltpu.sync_copy(zbuf, deg0.at[pl.ds(s * RPT, RPT)])

    @pl.when(c == 1)
    def _():
        pltpu.sync_copy(zbuf, deg1.at[pl.ds(s * RPT, RPT)])


@functools.partial(
    pl.kernel,
    out_type=(jax.ShapeDtypeStruct((NP, CW), F32),) * 2,
    mesh=_mesh,
    scratch_types=[
        pltpu.VMEM((2 * CH, K), jnp.int32),
        pltpu.VMEM((2 * CH, K), jnp.int32),
        pltpu.VMEM((RCH, CW), F32),
        pltpu.VMEM((K, CW), F32),
        pltpu.VMEM((K, CW), F32),
        pltpu.VMEM_SHARED((NP, CW), F32),
        pltpu.SemaphoreType.DMA,
        pltpu.SemaphoreType.DMA,
    ],
    compiler_params=pltpu.CompilerParams(use_tc_tiling_on_sc=False),
)
def _sc_agg(srcF, dstF, gA, gB, outA, outB, src_v, dst_v, zbuf, rows0, rows1,
            acc, sem0, sem1):
    c = lax.axis_index("c")
    s = lax.axis_index("s")

    pltpu.sync_copy(srcF.at[s], src_v.at[pl.ds(0, CH)])
    pltpu.sync_copy(srcF.at[s + NS], src_v.at[pl.ds(CH, CH)])
    pltpu.sync_copy(dstF.at[s], dst_v.at[pl.ds(0, CH)])
    pltpu.sync_copy(dstF.at[s + NS], dst_v.at[pl.ds(CH, CH)])

    _zero_fill(zbuf, RCH, CW)
    for k in range(RPT // RCH):
        pltpu.sync_copy(zbuf, acc.at[pl.ds(s * RPT + k * RCH, RCH)])
    plsc.subcore_barrier()

    def run(g_ref):
        pltpu.async_copy(g_ref.at[src_v.at[0]], rows0, sem0)

        def pair(k, _):
            j0 = 2 * k
            j1 = j0 + 1
            pltpu.async_copy(g_ref.at[src_v.at[j1]], rows1, sem1)
            pltpu.make_async_copy(g_ref.at[src_v.at[j0]], rows0, sem0).wait()
            pltpu.sync_copy(rows0, acc.at[dst_v.at[j0]], add=True)

            @pl.when(j1 + 1 < 2 * CH)
            def _():
                pltpu.async_copy(g_ref.at[src_v.at[j1 + 1]], rows0, sem0)

            pltpu.make_async_copy(g_ref.at[src_v.at[j1]], rows1, sem1).wait()
            pltpu.sync_copy(rows1, acc.at[dst_v.at[j1]], add=True)
            return 0

        lax.fori_loop(0, CH, pair, 0)

    @pl.when(c == 0)
    def _():
        run(gA)

    @pl.when(c == 1)
    def _():
        run(gB)

    plsc.subcore_barrier()

    for k in range(RPT // RCH):
        sl = pl.ds(s * RPT + k * RCH, RCH)
        pltpu.sync_copy(acc.at[sl], zbuf)

        @pl.when(c == 0)
        def _():
            pltpu.sync_copy(zbuf, outA.at[sl])

        @pl.when(c == 1)
        def _():
            pltpu.sync_copy(zbuf, outB.at[sl])


_BN = 1000


def _dinv(d0, d1):
    deg = d0[:, 0:1] + d1[:, 0:1] + 1.0
    return lax.rsqrt(deg)


def _tc_a_body(d0, d1, x_ref, w_ref, g0, g1, g2, g3):
    dinv = _dinv(d0[...], d1[...])
    h = jnp.dot(x_ref[...], w_ref[...], preferred_element_type=F32)
    g = h * dinv
    g0[...] = g[:, 0 * CW:1 * CW]
    g1[...] = g[:, 1 * CW:2 * CW]
    g2[...] = g[:, 2 * CW:3 * CW]
    g3[...] = g[:, 3 * CW:4 * CW]


def _tc_a(deg0, deg1, x, W1):
    blk = lambda w: pl.BlockSpec((_BN, w), lambda i: (i, 0))
    return pl.pallas_call(
        _tc_a_body,
        grid=(N // _BN,),
        in_specs=[
            blk(16), blk(16), blk(C2),
            pl.BlockSpec((C2, C1), lambda i: (0, 0)),
        ],
        out_specs=[blk(CW)] * 4,
        out_shape=(jax.ShapeDtypeStruct((N, CW), F32),) * 4,
    )(deg0, deg1, x, W1)


def _tc_b_body(d0, d1, s0, s1, s2, s3, g0, g1, g2, g3, b1, w2, o0, o1):
    dinv = _dinv(d0[...], d1[...])
    t = jnp.concatenate(
        [s0[...] + g0[...], s1[...] + g1[...],
         s2[...] + g2[...], s3[...] + g3[...]], axis=1)
    out1 = jnp.maximum(t * dinv + b1[...], 0.0)
    gg = jnp.dot(out1, w2[...], preferred_element_type=F32) * dinv
    o0[...] = gg[:, :CW]
    o1[...] = gg[:, CW:]


def _tc_b(deg0, deg1, s_parts, g_parts, b1, W2):
    blk = lambda w: pl.BlockSpec((_BN, w), lambda i: (i, 0))
    return pl.pallas_call(
        _tc_b_body,
        grid=(N // _BN,),
        in_specs=[blk(16), blk(16)] + [blk(CW)] * 8 + [
            pl.BlockSpec((1, C1), lambda i: (0, 0)),
            pl.BlockSpec((C1, C2), lambda i: (0, 0)),
        ],
        out_specs=[blk(CW)] * 2,
        out_shape=(jax.ShapeDtypeStruct((N, CW), F32),) * 2,
    )(deg0, deg1, *s_parts, *g_parts, b1, W2)


def _tc_c_body(d0, d1, t0, t1, g0, g1, b2, out_ref):
    dinv = _dinv(d0[...], d1[...])
    t = jnp.concatenate([t0[...] + g0[...], t1[...] + g1[...]], axis=1)
    out_ref[...] = jnp.maximum(t * dinv + b2[...], 0.0)


def _tc_c(deg0, deg1, t0, t1, g0, g1, b2):
    blk = lambda w: pl.BlockSpec((_BN, w), lambda i: (i, 0))
    return pl.pallas_call(
        _tc_c_body,
        grid=(N // _BN,),
        in_specs=[blk(16), blk(16), blk(CW), blk(CW), blk(CW), blk(CW),
                  pl.BlockSpec((1, C2), lambda i: (0, 0))],
        out_specs=blk(C2),
        out_shape=jax.ShapeDtypeStruct((N, C2), F32),
    )(deg0, deg1, t0, t1, g0, g1, b2)


def kernel(x, A, W1, b1, W2, b2):
    src = A[0].astype(jnp.int32)
    dst = A[1].astype(jnp.int32)
    srcF = src.reshape(BLK, CH, K)
    dstF = dst.reshape(BLK, CH, K)

    deg0, deg1 = _sc_deg(dstF)
    g1_0, g1_1, g1_2, g1_3 = _tc_a(deg0, deg1, x, W1)
    s0, s1 = _sc_agg(srcF, dstF, g1_0, g1_1)
    s2, s3 = _sc_agg(srcF, dstF, g1_2, g1_3)
    g2_0, g2_1 = _tc_b(deg0, deg1, (s0, s1, s2, s3),
                       (g1_0, g1_1, g1_2, g1_3), b1.reshape(1, C1), W2)
    t0, t1 = _sc_agg(srcF, dstF, g2_0, g2_1)
    return _tc_c(deg0, deg1, t0, t1, g2_0, g2_1, b2.reshape(1, C2))
